# baseline (speedup 1.0000x reference)
import functools

import jax
import jax.numpy as jnp
from jax.experimental import pallas as pl
from jax.experimental.pallas import tpu as pltpu

EMB = 128
OUT_EMB = 256
INT_EMB = 64
NSPH = 7
NRAD = 6
CUTOFF = 5.0
PEXP = 5

ETILE = 2000


def _silu(x):
    return x * jax.lax.logistic(x)


def _wspec(shape):
    return pl.BlockSpec(shape, lambda i: (0,) * len(shape))


def _rowspec(cols):
    return pl.BlockSpec((ETILE, cols), lambda i: (i, 0))


def _blockA_body(x_ref, rbf_ref, Wji_ref, bji_ref, Wkj_ref, bkj_ref, RB_ref,
                 down_ref, e1_ref, e2d_ref):
    x = x_ref[...]
    e1_ref[...] = _silu(x @ Wji_ref[...] + bji_ref[...])
    e2 = _silu(x @ Wkj_ref[...] + bkj_ref[...]) * (rbf_ref[...] @ RB_ref[...])
    e2d_ref[...] = _silu(e2 @ down_ref[...])


def _blockA(x, rbf8, b):
    n = x.shape[0]
    RB = jnp.pad(b['rbf1'] @ b['rbf2'], ((0, 2), (0, 0)))
    return pl.pallas_call(
        _blockA_body,
        grid=(n // ETILE,),
        in_specs=[_rowspec(EMB), _rowspec(8),
                  _wspec((EMB, EMB)), _wspec((1, EMB)),
                  _wspec((EMB, EMB)), _wspec((1, EMB)),
                  _wspec((8, EMB)), _wspec((EMB, INT_EMB))],
        out_specs=[_rowspec(EMB), _rowspec(INT_EMB)],
        out_shape=[jax.ShapeDtypeStruct((n, EMB), jnp.float32),
                   jax.ShapeDtypeStruct((n, INT_EMB), jnp.float32)],
    )(x, rbf8, b['Wji'], b['bji'].reshape(1, -1), b['Wkj'],
      b['bkj'].reshape(1, -1), RB, b['down'])


def _blockB_body(agg_ref, e1_ref, x_ref, rbf_ref, up_ref,
                 bW1, bb1, bW2, bb2, Wfin, bfin,
                 aW1, ab1, aW2, ab2, aW3, ab3, aW4, ab4,
                 Wrbf, xn_ref, gx_ref):
    u = _silu(agg_ref[...] @ up_ref[...])
    hm = e1_ref[...] + u
    hm = hm + _silu(_silu(hm @ bW1[...] + bb1[...]) @ bW2[...] + bb2[...])
    hn = _silu(hm @ Wfin[...] + bfin[...]) + x_ref[...]
    hn = hn + _silu(_silu(hn @ aW1[...] + ab1[...]) @ aW2[...] + ab2[...])
    hn = hn + _silu(_silu(hn @ aW3[...] + ab3[...]) @ aW4[...] + ab4[...])
    xn_ref[...] = hn
    gx_ref[...] = (rbf_ref[...] @ Wrbf[...]) * hn


def _blockB(agg, e1, x, rbf8, b, o):
    n = x.shape[0]
    (bW1, bb1, bW2, bb2), = b['before']
    (aW1, ab1, aW2, ab2), (aW3, ab3, aW4, ab4) = b['after']
    Wrbf8 = jnp.pad(o['Wrbf'], ((0, 2), (0, 0)))
    r = lambda v: v.reshape(1, -1)
    return pl.pallas_call(
        _blockB_body,
        grid=(n // ETILE,),
        in_specs=[_rowspec(INT_EMB), _rowspec(EMB), _rowspec(EMB), _rowspec(8),
                  _wspec((INT_EMB, EMB)),
                  _wspec((EMB, EMB)), _wspec((1, EMB)),
                  _wspec((EMB, EMB)), _wspec((1, EMB)),
                  _wspec((EMB, EMB)), _wspec((1, EMB)),
                  _wspec((EMB, EMB)), _wspec((1, EMB)),
                  _wspec((EMB, EMB)), _wspec((1, EMB)),
                  _wspec((EMB, EMB)), _wspec((1, EMB)),
                  _wspec((EMB, EMB)), _wspec((1, EMB)),
                  _wspec((8, EMB))],
        out_specs=[_rowspec(EMB), _rowspec(EMB)],
        out_shape=[jax.ShapeDtypeStruct((n, EMB), jnp.float32),
                   jax.ShapeDtypeStruct((n, EMB), jnp.float32)],
    )(agg, e1, x, rbf8, b['up'],
      bW1, r(bb1), bW2, r(bb2), b['Wfin'], r(b['bfin']),
      aW1, r(ab1), aW2, r(ab2), aW3, r(ab3), aW4, r(ab4), Wrbf8)


def _envelope(x):
    p = PEXP
    a = -(p + 1) * (p + 2) / 2.0
    b = p * (p + 2.0)
    c = -p * (p + 1) / 2.0
    xs = jnp.clip(x, 1e-9, None)
    env = 1.0 / xs + a * xs ** (p - 1) + b * xs ** p + c * xs ** (p + 1)
    return jnp.where(x < 1.0, env, 0.0)


def _radial(x):
    freqs = jnp.pi * jnp.arange(1, NRAD + 1, dtype=jnp.float32)
    return _envelope(x)[:, None] * jnp.sin(freqs[None, :] * x[:, None])


def kernel(Z, R, batch_seg, idnb_i, idnb_j, id_expand_kj, id_reduce_ji,
           id3dnb_i, id3dnb_j, id3dnb_k, params):
    n_atoms = Z.shape[0]
    n_graph = 512

    Ri = R[idnb_i]
    Rj = R[idnb_j]
    Dij = jnp.sqrt(jnp.maximum(jnp.sum((Ri - Rj) ** 2, -1), 1e-12))
    rbf = _radial(Dij / CUTOFF)
    rbf8 = jnp.pad(rbf, ((0, 0), (0, 2)))

    R1 = R[id3dnb_j] - R[id3dnb_i]
    R2 = R[id3dnb_k] - R[id3dnb_j]
    xdot = jnp.sum(R1 * R2, -1)
    ycr = jnp.sqrt(jnp.sum(jnp.cross(R1, R2) ** 2, -1) + 1e-9)
    angles = jnp.arctan2(ycr, xdot)
    rad_t = _radial((Dij / CUTOFF)[id_expand_kj])
    ls = jnp.arange(NSPH, dtype=jnp.float32)
    angular = jnp.cos(ls[None, :] * angles[:, None])
    sbf = (angular[:, :, None] * rad_t[:, None, :]).reshape(-1, NSPH * NRAD)

    h = params['z_emb'][Z]
    rbf_e = _silu(rbf @ params['emb_rbf_W'] + params['emb_rbf_b'])
    x = _silu(jnp.concatenate([h[idnb_i], h[idnb_j], rbf_e], -1)
              @ params['emb_cat_W'] + params['emb_cat_b'])

    def atom_chain(o, t):
        t = t @ o['Wup']
        for (W, b) in o['dense']:
            t = _silu(t @ W + b)
        return t @ o['Wout']

    o0 = params['out'][0]
    gx = (rbf @ o0['Wrbf']) * x
    t0 = jax.ops.segment_sum(gx, idnb_i, num_segments=n_atoms)
    P_atom = atom_chain(o0, t0)

    for i in range(3):
        b = params['int'][i]
        o = params['out'][i + 1]
        e1, e2d = _blockA(x, rbf8, b)
        SB = b['sbf1'] @ b['sbf2']
        m = e2d[id_expand_kj] * (sbf @ SB)
        agg = jax.ops.segment_sum(m, id_reduce_ji, num_segments=x.shape[0])
        x, gx = _blockB(agg, e1, x, rbf8, b, o)
        t = jax.ops.segment_sum(gx, idnb_i, num_segments=n_atoms)
        P_atom = P_atom + atom_chain(o, t)

    energy = jax.ops.segment_sum(P_atom, batch_seg, num_segments=n_graph)
    return energy
